# baseline (speedup 1.0000x reference)
import functools

import numpy as np
import jax
import jax.numpy as jnp
from jax import lax
from jax.experimental import pallas as pl
from jax.experimental.pallas import tpu as pltpu, tpu_sc as plsc

VOCAB = 100000
D_MODEL = 1024
BATCH = 4
SEQ = 4096

_NC = 2
_NS = 16
_NW = _NC * _NS
_POS_PER_W = SEQ // _NW
_C = 32
_K = _POS_PER_W // _C
_STEPS = _K * BATCH
_H = _C // 2
_NH = _POS_PER_W // _H
_LANES = 16
_VECS = D_MODEL // _LANES


def _pe_table() -> np.ndarray:
    pos = np.arange(SEQ, dtype=np.float32)[:, None]
    two_i = np.arange(0, D_MODEL, 2, dtype=np.float32)
    div = np.power(10000.0, two_i / D_MODEL)
    pe = np.zeros((SEQ, D_MODEL), dtype=np.float32)
    pe[:, 0::2] = np.sin(pos / div)
    pe[:, 1::2] = np.cos(pos / div)
    return pe


_PE = _pe_table()


@functools.partial(
    pl.kernel,
    mesh=plsc.VectorSubcoreMesh(core_axis_name="c", subcore_axis_name="s"),
    out_type=jax.ShapeDtypeStruct((BATCH, SEQ, D_MODEL), jnp.float32),
    scratch_types=(
        [pltpu.VMEM((BATCH, _POS_PER_W), jnp.int32)]
        + [pltpu.VMEM((_H, D_MODEL), jnp.float32)] * 3
        + [pltpu.VMEM((_C, D_MODEL), jnp.float32)] * 2
        + [pltpu.SemaphoreType.DMA] * 7
    ),
)
def _emb_kernel(table_hbm, x_hbm, pe_hbm, out_hbm,
                idx_all, pe0, pe1, pe2, tok0, tok1,
                gs0, gs1, ss0, ss1, ps0, ps1, ps2):
    pe_v = (pe0, pe1, pe2)
    ps = (ps0, ps1, ps2)
    tok = (tok0, tok1)
    gs = (gs0, gs1)
    ss = (ss0, ss1)

    wid = lax.axis_index("s") * _NC + lax.axis_index("c")
    pos0 = wid * _POS_PER_W

    for b in range(BATCH):
        pltpu.sync_copy(x_hbm.at[b, pl.ds(pos0, _POS_PER_W)], idx_all.at[b])

    def pe_fetch(h):
        return pltpu.async_copy(
            pe_hbm.at[pl.ds(pos0 + h * _H, _H)], pe_v[h % 3], ps[h % 3])

    pe_descs = {0: pe_fetch(0), 1: pe_fetch(1)}

    gather_descs = [None, None]
    store_descs = [None, None]

    for s in range(_STEPS + 1):
        if s < _STEPS:
            buf = s % 2
            k, b = s // BATCH, s % BATCH
            if store_descs[buf] is not None:
                store_descs[buf].wait()
            gather_descs[buf] = pltpu.async_copy(
                table_hbm.at[idx_all.at[b, pl.ds(k * _C, _C)]],
                tok[buf], gs[buf])

        if s >= 1:
            cs = s - 1
            cbuf = cs % 2
            ck, cb = cs // BATCH, cs % BATCH
            if cs % BATCH == 0:
                pe_descs.pop(2 * ck).wait()
                pe_descs.pop(2 * ck + 1).wait()
                if 2 * ck + 2 < _NH:
                    pe_descs[2 * ck + 2] = pe_fetch(2 * ck + 2)
            gather_descs[cbuf].wait()
            tk = tok[cbuf]

            for half in range(2):
                pk = pe_v[(2 * ck + half) % 3]
                row_off = half * _H

                def row_body(i, _, tk=tk, pk=pk, row_off=row_off):
                    for j in range(_VECS):
                        sl = pl.ds(j * _LANES, _LANES)
                        plsc.addupdate(tk.at[row_off + i, sl], pk[i, sl])
                    return 0

                lax.fori_loop(0, _H, row_body, 0)

            store_descs[cbuf] = pltpu.async_copy(
                tk, out_hbm.at[cb, pl.ds(pos0 + ck * _C, _C)], ss[cbuf])

            if cs % BATCH == BATCH - 1 and 2 * ck + 3 < _NH:
                pe_descs[2 * ck + 3] = pe_fetch(2 * ck + 3)

    for buf in range(2):
        if store_descs[buf] is not None:
            store_descs[buf].wait()


def kernel(x, token_table):
    x = x.astype(jnp.int32)
    pe = jnp.asarray(_PE)
    return _emb_kernel(token_table, x, pe)


# speedup vs baseline: 1.1234x; 1.1234x over previous
import functools

import numpy as np
import jax
import jax.numpy as jnp
from jax import lax
from jax.experimental import pallas as pl
from jax.experimental.pallas import tpu as pltpu, tpu_sc as plsc

VOCAB = 100000
D_MODEL = 1024
BATCH = 4
SEQ = 4096

_NC = 2
_NS = 16
_NW = _NC * _NS
_POS_PER_W = SEQ // _NW
_C = 32
_K = _POS_PER_W // _C
_STEPS = _K * BATCH
_LANES = 16
_VECS = D_MODEL // _LANES


def _pe_table() -> np.ndarray:
    pos = np.arange(SEQ, dtype=np.float32)[:, None]
    two_i = np.arange(0, D_MODEL, 2, dtype=np.float32)
    div = np.power(10000.0, two_i / D_MODEL)
    pe = np.zeros((SEQ, D_MODEL), dtype=np.float32)
    pe[:, 0::2] = np.sin(pos / div)
    pe[:, 1::2] = np.cos(pos / div)
    return pe


_PE = _pe_table()


@functools.partial(
    pl.kernel,
    mesh=plsc.VectorSubcoreMesh(core_axis_name="c", subcore_axis_name="s"),
    out_type=jax.ShapeDtypeStruct((BATCH, SEQ, D_MODEL), jnp.float32),
    scratch_types=(
        [pltpu.VMEM((BATCH, _POS_PER_W), jnp.int32)]
        + [pltpu.VMEM((_C, D_MODEL), jnp.float32)]
        + [pltpu.VMEM((_C, D_MODEL), jnp.float32)] * 2
        + [pltpu.SemaphoreType.DMA] * 4
    ),
)
def _emb_kernel(table_hbm, x_hbm, pe_hbm, out_hbm,
                idx_all, pe_v, tok0, tok1, gs0, gs1, ss0, ss1):
    tok = (tok0, tok1)
    gs = (gs0, gs1)
    ss = (ss0, ss1)

    wid = lax.axis_index("s") * _NC + lax.axis_index("c")
    pos0 = wid * _POS_PER_W

    for b in range(BATCH):
        pltpu.sync_copy(x_hbm.at[b, pl.ds(pos0, _POS_PER_W)], idx_all.at[b])
    pltpu.sync_copy(pe_hbm.at[pl.ds(pos0, _C)], pe_v)

    gather_descs = [None, None]
    store_descs = [None, None]

    for s in range(_STEPS + 1):
        if s < _STEPS:
            buf = s % 2
            k, b = s // BATCH, s % BATCH
            if store_descs[buf] is not None:
                store_descs[buf].wait()
            gather_descs[buf] = pltpu.async_copy(
                table_hbm.at[idx_all.at[b, pl.ds(k * _C, _C)]],
                tok[buf], gs[buf])

        if s >= 1:
            cs = s - 1
            cbuf = cs % 2
            ck, cb = cs // BATCH, cs % BATCH
            gather_descs[cbuf].wait()
            tk = tok[cbuf]

            def row_body(i, _, tk=tk):
                for j in range(_VECS):
                    sl = pl.ds(j * _LANES, _LANES)
                    plsc.addupdate(tk.at[i, sl], pe_v[i, sl])
                return 0

            lax.fori_loop(0, _C, row_body, 0)

            store_descs[cbuf] = pltpu.async_copy(
                tk, out_hbm.at[cb, pl.ds(pos0 + ck * _C, _C)], ss[cbuf])

            if s % BATCH == 0 and s < _STEPS:
                pltpu.sync_copy(pe_hbm.at[pl.ds(pos0 + (s // BATCH) * _C, _C)],
                                pe_v)

    for buf in range(2):
        if store_descs[buf] is not None:
            store_descs[buf].wait()


def kernel(x, token_table):
    x = x.astype(jnp.int32)
    pe = jnp.asarray(_PE)
    return _emb_kernel(token_table, x, pe)


# speedup vs baseline: 1.1747x; 1.0457x over previous
import functools

import numpy as np
import jax
import jax.numpy as jnp
from jax import lax
from jax.experimental import pallas as pl
from jax.experimental.pallas import tpu as pltpu, tpu_sc as plsc

VOCAB = 100000
D_MODEL = 1024
BATCH = 4
SEQ = 4096

_NC = 2
_NS = 16
_NW = _NC * _NS
_POS_PER_W = SEQ // _NW
_C = 32
_K = _POS_PER_W // _C
_STEPS = _K * BATCH
_LANES = 16
_VECS = D_MODEL // _LANES


def _pe_table() -> np.ndarray:
    pos = np.arange(SEQ, dtype=np.float32)[:, None]
    two_i = np.arange(0, D_MODEL, 2, dtype=np.float32)
    div = np.power(10000.0, two_i / D_MODEL)
    pe = np.zeros((SEQ, D_MODEL), dtype=np.float32)
    pe[:, 0::2] = np.sin(pos / div)
    pe[:, 1::2] = np.cos(pos / div)
    return pe


_PE = _pe_table()


@functools.partial(
    pl.kernel,
    mesh=plsc.VectorSubcoreMesh(core_axis_name="c", subcore_axis_name="s"),
    out_type=jax.ShapeDtypeStruct((BATCH, SEQ, D_MODEL), jnp.float32),
    scratch_types=(
        [pltpu.VMEM((BATCH, _POS_PER_W), jnp.int32)]
        + [pltpu.VMEM((_C, D_MODEL), jnp.float32)]
        + [pltpu.VMEM((_C, D_MODEL), jnp.float32)] * 2
        + [pltpu.SemaphoreType.DMA] * 4
    ),
)
def _emb_kernel(table_hbm, x_hbm, pe_hbm, out_hbm,
                idx_all, pe_v, tok0, tok1, gs0, gs1, ss0, ss1):
    tok = (tok0, tok1)
    gs = (gs0, gs1)
    ss = (ss0, ss1)

    wid = lax.axis_index("s") * _NC + lax.axis_index("c")
    pos0 = wid * _POS_PER_W

    for b in range(BATCH):
        pltpu.sync_copy(x_hbm.at[b, pl.ds(pos0, _POS_PER_W)], idx_all.at[b])
    pltpu.sync_copy(pe_hbm.at[pl.ds(pos0, _C)], pe_v)

    gather_descs = [None, None]
    store_descs = [None, None]

    for s in range(_STEPS + 1):
        if s < _STEPS:
            buf = s % 2
            k, b = s // BATCH, s % BATCH
            if store_descs[buf] is not None:
                store_descs[buf].wait()
            gather_descs[buf] = pltpu.async_copy(
                table_hbm.at[idx_all.at[b, pl.ds(k * _C, _C)]],
                tok[buf], gs[buf])

        if s >= 1:
            cs = s - 1
            cbuf = cs % 2
            ck, cb = cs // BATCH, cs % BATCH
            gather_descs[cbuf].wait()
            tk = tok[cbuf]

            @plsc.parallel_loop(0, _C)
            def row_body(i, tk=tk):
                for j in range(_VECS):
                    sl = pl.ds(j * _LANES, _LANES)
                    plsc.addupdate(tk.at[i, sl], pe_v[i, sl])

            store_descs[cbuf] = pltpu.async_copy(
                tk, out_hbm.at[cb, pl.ds(pos0 + ck * _C, _C)], ss[cbuf])

            if s % BATCH == 0 and s < _STEPS:
                pltpu.sync_copy(pe_hbm.at[pl.ds(pos0 + (s // BATCH) * _C, _C)],
                                pe_v)

    for buf in range(2):
        if store_descs[buf] is not None:
            store_descs[buf].wait()


def kernel(x, token_table):
    x = x.astype(jnp.int32)
    pe = jnp.asarray(_PE)
    return _emb_kernel(token_table, x, pe)


# speedup vs baseline: 1.1756x; 1.0007x over previous
import functools

import numpy as np
import jax
import jax.numpy as jnp
from jax import lax
from jax.experimental import pallas as pl
from jax.experimental.pallas import tpu as pltpu, tpu_sc as plsc

VOCAB = 100000
D_MODEL = 1024
BATCH = 4
SEQ = 4096

_NC = 2
_NS = 16
_NW = _NC * _NS
_POS_PER_W = SEQ // _NW
_C = 32
_K = _POS_PER_W // _C
_STEPS = _K * BATCH
_LANES = 16
_VECS = D_MODEL // _LANES


def _pe_table() -> np.ndarray:
    pos = np.arange(SEQ, dtype=np.float32)[:, None]
    two_i = np.arange(0, D_MODEL, 2, dtype=np.float32)
    div = np.power(10000.0, two_i / D_MODEL)
    pe = np.zeros((SEQ, D_MODEL), dtype=np.float32)
    pe[:, 0::2] = np.sin(pos / div)
    pe[:, 1::2] = np.cos(pos / div)
    return pe


_PE = _pe_table()


@functools.partial(
    pl.kernel,
    mesh=plsc.VectorSubcoreMesh(core_axis_name="c", subcore_axis_name="s"),
    out_type=jax.ShapeDtypeStruct((BATCH, SEQ, D_MODEL), jnp.float32),
    scratch_types=(
        [pltpu.VMEM((BATCH, _POS_PER_W), jnp.int32)]
        + [pltpu.VMEM((_C, D_MODEL), jnp.float32)]
        + [pltpu.VMEM((_C, D_MODEL), jnp.float32)] * 2
        + [pltpu.SemaphoreType.DMA] * 4
    ),
)
def _emb_kernel(table_hbm, x_hbm, pe_hbm, out_hbm,
                idx_all, pe_v, tok0, tok1, gs0, gs1, ss0, ss1):
    tok = (tok0, tok1)
    gs = (gs0, gs1)
    ss = (ss0, ss1)

    wid = lax.axis_index("s") * _NC + lax.axis_index("c")
    pos0 = wid * _POS_PER_W

    for b in range(BATCH):
        pltpu.sync_copy(x_hbm.at[b, pl.ds(pos0, _POS_PER_W)], idx_all.at[b])
    pltpu.sync_copy(pe_hbm.at[pl.ds(pos0, _C)], pe_v)

    gather_descs = [None, None]
    store_descs = [None, None]

    for s in range(_STEPS + 1):
        if s < _STEPS:
            buf = s % 2
            k, b = s // BATCH, s % BATCH
            if store_descs[buf] is not None:
                for d in store_descs[buf]:
                    d.wait()
            gather_descs[buf] = pltpu.async_copy(
                table_hbm.at[idx_all.at[b, pl.ds(k * _C, _C)]],
                tok[buf], gs[buf])

        if s >= 1:
            cs = s - 1
            cbuf = cs % 2
            ck, cb = cs // BATCH, cs % BATCH
            gather_descs[cbuf].wait()
            tk = tok[cbuf]

            @plsc.parallel_loop(0, _C)
            def row_body(i, tk=tk):
                for j in range(_VECS):
                    sl = pl.ds(j * _LANES, _LANES)
                    plsc.addupdate(tk.at[i, sl], pe_v[i, sl])

            store_descs[cbuf] = [pltpu.async_copy(
                tk, out_hbm.at[cb, pl.ds(pos0 + ck * _C, _C)], ss[cbuf])]

            if s % BATCH == 0 and s < _STEPS:
                pltpu.sync_copy(pe_hbm.at[pl.ds(pos0 + (s // BATCH) * _C, _C)],
                                pe_v)

    for buf in range(2):
        if store_descs[buf] is not None:
            for d in store_descs[buf]:
                d.wait()


def kernel(x, token_table):
    x = x.astype(jnp.int32)
    pe = jnp.asarray(_PE)
    return _emb_kernel(token_table, x, pe)
